# Initial kernel scaffold; baseline (speedup 1.0000x reference)
import functools

import jax
import jax.numpy as jnp
from jax import lax
from jax.experimental import pallas as pl
from jax.experimental.pallas import tpu as pltpu
from jax.experimental.pallas import tpu_sc as plsc

_N = 10000
_NP = 10240
_D = 128
_CHUNK = 128
_CPT = 79
_EP = 32 * _CPT * _CHUNK
_RPT = _NP // 16
_NB = _NP // 128


def _sc_mesh():
    return plsc.VectorSubcoreMesh(core_axis_name="c", subcore_axis_name="s")


def _hist_body(dst_hbm, ones_hbm, zero_hbm, out_hbm, didx, ones_v, stage, deg):
    c = lax.axis_index("c")
    s = lax.axis_index("s")
    w = c * 16 + s
    pltpu.sync_copy(zero_hbm, stage)
    pltpu.sync_copy(stage, deg.at[pl.ds(s * _RPT, _RPT)])
    pltpu.sync_copy(ones_hbm, ones_v)
    plsc.subcore_barrier()

    def step(j, carry):
        base = (w * _CPT + j) * _CHUNK
        pltpu.sync_copy(dst_hbm.at[pl.ds(base, _CHUNK)], didx)
        pltpu.sync_copy(ones_v, deg.at[didx], add=True)
        return carry

    lax.fori_loop(0, _CPT, step, 0)
    plsc.subcore_barrier()
    pltpu.sync_copy(deg.at[pl.ds(s * _RPT, _RPT)], stage)
    pltpu.sync_copy(stage, out_hbm.at[pl.ds(c * _NP + s * _RPT, _RPT)])


def _sc_hist(dst_pad, ones_rows, zero_rows):
    return pl.kernel(
        _hist_body,
        out_type=jax.ShapeDtypeStruct((2 * _NP, 16), jnp.float32),
        mesh=_sc_mesh(),
        scratch_types=[
            pltpu.VMEM((_CHUNK,), jnp.int32),
            pltpu.VMEM((_CHUNK, 16), jnp.float32),
            pltpu.VMEM((_RPT, 16), jnp.float32),
            pltpu.VMEM_SHARED((_NP, 16), jnp.float32),
        ],
    )(dst_pad, ones_rows, zero_rows)


def _prop_body(h_hbm, src_hbm, dst_hbm, zero_hbm, out_hbm, sidx, didx, rows, acc):
    c = lax.axis_index("c")
    s = lax.axis_index("s")
    w = c * 16 + s
    pltpu.sync_copy(zero_hbm, rows)
    for k in range(_RPT // _CHUNK):
        pltpu.sync_copy(rows, acc.at[pl.ds(s * _RPT + k * _CHUNK, _CHUNK)])
    plsc.subcore_barrier()

    def step(j, carry):
        base = (w * _CPT + j) * _CHUNK
        pltpu.sync_copy(src_hbm.at[pl.ds(base, _CHUNK)], sidx)
        pltpu.sync_copy(dst_hbm.at[pl.ds(base, _CHUNK)], didx)
        pltpu.sync_copy(h_hbm.at[sidx], rows)
        pltpu.sync_copy(rows, acc.at[didx], add=True)
        return carry

    lax.fori_loop(0, _CPT, step, 0)
    plsc.subcore_barrier()
    for k in range(_RPT // _CHUNK):
        r = s * _RPT + k * _CHUNK
        pltpu.sync_copy(acc.at[pl.ds(r, _CHUNK)], rows)
        pltpu.sync_copy(rows, out_hbm.at[pl.ds(c * _NP + r, _CHUNK)])


def _sc_propagate(h, src_pad, dst_pad, zero_rows):
    return pl.kernel(
        _prop_body,
        out_type=jax.ShapeDtypeStruct((2 * _NP, _D), jnp.float32),
        mesh=_sc_mesh(),
        scratch_types=[
            pltpu.VMEM((_CHUNK,), jnp.int32),
            pltpu.VMEM((_CHUNK,), jnp.int32),
            pltpu.VMEM((_CHUNK, _D), jnp.float32),
            pltpu.VMEM_SHARED((_NP, _D), jnp.float32),
        ],
    )(h, src_pad, dst_pad, zero_rows)


def _tc_first_body(dega_ref, degb_ref, x_ref, w_ref, dinv_ref, h_ref):
    i = pl.program_id(0)
    deg = dega_ref[:, 0:1] + degb_ref[:, 0:1] + 1.0
    row = lax.broadcasted_iota(jnp.int32, (128, 1), 0) + i * 128
    dinv = jnp.where(row < _N, lax.rsqrt(deg), 0.0)
    dinv_b = jnp.broadcast_to(dinv, (128, _D))
    dinv_ref[...] = dinv_b
    h_ref[...] = dinv_b * jnp.dot(x_ref[...], w_ref[...],
                                  preferred_element_type=jnp.float32)


def _tc_first(degp, xp, W0):
    return pl.pallas_call(
        _tc_first_body,
        grid=(_NB,),
        in_specs=[
            pl.BlockSpec((128, 16), lambda i: (i, 0)),
            pl.BlockSpec((128, 16), lambda i: (i + _NB, 0)),
            pl.BlockSpec((128, _D), lambda i: (i, 0)),
            pl.BlockSpec((_D, _D), lambda i: (0, 0)),
        ],
        out_specs=[pl.BlockSpec((128, _D), lambda i: (i, 0))] * 2,
        out_shape=[jax.ShapeDtypeStruct((_NP, _D), jnp.float32)] * 2,
    )(degp, degp, xp, W0)


def _tc_mid_body(p0_ref, p1_ref, h_ref, dinv_ref, b_ref, w_ref, o_ref):
    pre = dinv_ref[...] * (p0_ref[...] + p1_ref[...] + h_ref[...]) + b_ref[...]
    a = jnp.tanh(pre)
    o_ref[...] = dinv_ref[...] * jnp.dot(a, w_ref[...],
                                         preferred_element_type=jnp.float32)


def _tc_mid(p, h_prev, dinv, b, W):
    return pl.pallas_call(
        _tc_mid_body,
        grid=(_NB,),
        in_specs=[
            pl.BlockSpec((128, _D), lambda i: (i, 0)),
            pl.BlockSpec((128, _D), lambda i: (i + _NB, 0)),
            pl.BlockSpec((128, _D), lambda i: (i, 0)),
            pl.BlockSpec((128, _D), lambda i: (i, 0)),
            pl.BlockSpec((1, _D), lambda i: (0, 0)),
            pl.BlockSpec((_D, _D), lambda i: (0, 0)),
        ],
        out_specs=pl.BlockSpec((128, _D), lambda i: (i, 0)),
        out_shape=jax.ShapeDtypeStruct((_NP, _D), jnp.float32),
    )(p, p, h_prev, dinv, b.reshape(1, _D), W)


def _tc_final_body(p0_ref, p1_ref, h_ref, dinv_ref, b_ref, o_ref):
    pre = dinv_ref[...] * (p0_ref[...] + p1_ref[...] + h_ref[...]) + b_ref[...]
    o_ref[...] = jnp.tanh(pre)


def _tc_final(p, h_prev, dinv, b):
    return pl.pallas_call(
        _tc_final_body,
        grid=(_NB,),
        in_specs=[
            pl.BlockSpec((128, _D), lambda i: (i, 0)),
            pl.BlockSpec((128, _D), lambda i: (i + _NB, 0)),
            pl.BlockSpec((128, _D), lambda i: (i, 0)),
            pl.BlockSpec((128, _D), lambda i: (i, 0)),
            pl.BlockSpec((1, _D), lambda i: (0, 0)),
        ],
        out_specs=pl.BlockSpec((128, _D), lambda i: (i, 0)),
        out_shape=jax.ShapeDtypeStruct((_NP, _D), jnp.float32),
    )(p, p, h_prev, dinv, b.reshape(1, _D))


def kernel(x, edge_index, W0, b0, W1, b1, W2, b2, W3, b3):
    n = x.shape[0]
    e = edge_index.shape[1]
    npad = _NP - n
    epad = _EP - e
    pad_src = jnp.arange(epad, dtype=jnp.int32) % n
    pad_dst = n + (jnp.arange(epad, dtype=jnp.int32) % npad)
    src_pad = jnp.concatenate([edge_index[0], pad_src])
    dst_pad = jnp.concatenate([edge_index[1], pad_dst])
    xp = jnp.pad(x, ((0, npad), (0, 0)))

    ones_rows = jnp.ones((_CHUNK, 16), jnp.float32)
    zero16 = jnp.zeros((_RPT, 16), jnp.float32)
    zero_rows = jnp.zeros((_CHUNK, _D), jnp.float32)

    degp = _sc_hist(dst_pad, ones_rows, zero16)
    dinv, h = _tc_first(degp, xp, W0)
    p = _sc_propagate(h, src_pad, dst_pad, zero_rows)
    h = _tc_mid(p, h, dinv, b0, W1)
    p = _sc_propagate(h, src_pad, dst_pad, zero_rows)
    h = _tc_mid(p, h, dinv, b1, W2)
    p = _sc_propagate(h, src_pad, dst_pad, zero_rows)
    h = _tc_mid(p, h, dinv, b2, W3)
    p = _sc_propagate(h, src_pad, dst_pad, zero_rows)
    out = _tc_final(p, h, dinv, b3)
    return out[:n]


# speedup vs baseline: 10.5890x; 10.5890x over previous
import functools

import jax
import jax.numpy as jnp
from jax import lax
from jax.experimental import pallas as pl
from jax.experimental.pallas import tpu as pltpu
from jax.experimental.pallas import tpu_sc as plsc

_N = 10000
_NP = 10240
_D = 128
_CHUNK = 128
_CPT = 79
_EP = 32 * _CPT * _CHUNK
_RPT = _NP // 16
_NB = _NP // 128


def _sc_mesh():
    return plsc.VectorSubcoreMesh(core_axis_name="c", subcore_axis_name="s",
                                  num_cores=2, num_subcores=16)


_KPT = _RPT // _CHUNK


def _hist_body(dst_hbm, ones_hbm, zero_hbm, iota_hbm, out_hbm,
               didx, ones_v, zrows_v, gidx, rowbuf, deg):
    c = lax.axis_index("c")
    s = lax.axis_index("s")
    w = c * 16 + s
    pltpu.sync_copy(ones_hbm, ones_v)
    pltpu.sync_copy(zero_hbm, zrows_v)
    for k in range(_KPT):
        pltpu.sync_copy(
            iota_hbm.at[pl.ds(s * _RPT + k * _CHUNK, _CHUNK)], gidx)
        pltpu.sync_copy(zrows_v, deg.at[gidx])
    plsc.subcore_barrier()

    def step(j, carry):
        base = (w * _CPT + j) * _CHUNK
        pltpu.sync_copy(dst_hbm.at[pl.ds(base, _CHUNK)], didx)
        pltpu.sync_copy(ones_v, deg.at[didx], add=True)
        return carry

    lax.fori_loop(0, _CPT, step, 0)
    plsc.subcore_barrier()
    for k in range(_KPT):
        pltpu.sync_copy(
            iota_hbm.at[pl.ds(s * _RPT + k * _CHUNK, _CHUNK)], gidx)
        pltpu.sync_copy(deg.at[gidx], rowbuf)
        pltpu.sync_copy(
            rowbuf, out_hbm.at[pl.ds(c * _NP + s * _RPT + k * _CHUNK, _CHUNK)])


def _sc_hist(dst_pad, ones_rows, zero_rows, iota1):
    return pl.kernel(
        _hist_body,
        out_type=jax.ShapeDtypeStruct((2 * _NP, 16), jnp.float32),
        mesh=_sc_mesh(),
        scratch_types=[
            pltpu.VMEM((_CHUNK,), jnp.int32),
            pltpu.VMEM((_CHUNK, 16), jnp.float32),
            pltpu.VMEM((_CHUNK, 16), jnp.float32),
            pltpu.VMEM((_CHUNK,), jnp.int32),
            pltpu.VMEM((_CHUNK, 16), jnp.float32),
            pltpu.VMEM_SHARED((_NP, 16), jnp.float32),
        ],
    )(dst_pad, ones_rows, zero_rows, iota1)


def _prop_body(h_hbm, src_hbm, dst_hbm, zero_hbm, iota_hbm, out_hbm,
               sidx, didx, gidx, rows, zrows, acc, gsem):
    c = lax.axis_index("c")
    s = lax.axis_index("s")
    w = c * 16 + s
    pltpu.sync_copy(zero_hbm, zrows)
    for k in range(_KPT):
        pltpu.sync_copy(
            iota_hbm.at[pl.ds(s * _RPT + k * _CHUNK, _CHUNK)], gidx)
        pltpu.sync_copy(zrows, acc.at[gidx])
    plsc.subcore_barrier()

    def step(j, carry):
        base = (w * _CPT + j) * _CHUNK
        pltpu.sync_copy(src_hbm.at[pl.ds(base, _CHUNK)], sidx)
        pltpu.sync_copy(dst_hbm.at[pl.ds(base, _CHUNK)], didx)
        pltpu.sync_copy(h_hbm.at[sidx], rows)
        pltpu.sync_copy(rows, acc.at[didx], add=True)
        return carry

    lax.fori_loop(0, _CPT, step, 0)
    plsc.subcore_barrier()
    for k in range(_KPT):
        r = s * _RPT + k * _CHUNK
        pltpu.sync_copy(iota_hbm.at[pl.ds(r, _CHUNK)], gidx)
        pltpu.sync_copy(acc.at[gidx], rows)
        pltpu.sync_copy(rows, out_hbm.at[pl.ds(c * _NP + r, _CHUNK)])


def _sc_propagate(h, src_pad, dst_pad, zero_rows, iota1):
    return pl.kernel(
        _prop_body,
        out_type=jax.ShapeDtypeStruct((2 * _NP, _D), jnp.float32),
        mesh=_sc_mesh(),
        scratch_types=[
            pltpu.VMEM((_CHUNK,), jnp.int32),
            pltpu.VMEM((_CHUNK,), jnp.int32),
            pltpu.VMEM((_CHUNK,), jnp.int32),
            pltpu.VMEM((_CHUNK, _D), jnp.float32),
            pltpu.VMEM((_CHUNK, _D), jnp.float32),
            pltpu.VMEM_SHARED((_NP, _D), jnp.float32),
            pltpu.SemaphoreType.DMA,
        ],
    )(h, src_pad, dst_pad, zero_rows, iota1)


def _tc_first_body(dega_ref, degb_ref, x_ref, w_ref, dinv_ref, h_ref):
    i = pl.program_id(0)
    deg = dega_ref[:, 0:1] + degb_ref[:, 0:1] + 1.0
    row = lax.broadcasted_iota(jnp.int32, (128, 1), 0) + i * 128
    dinv = jnp.where(row < _N, lax.rsqrt(deg), 0.0)
    dinv_b = jnp.broadcast_to(dinv, (128, _D))
    dinv_ref[...] = dinv_b
    h_ref[...] = dinv_b * jnp.dot(x_ref[...], w_ref[...],
                                  preferred_element_type=jnp.float32)


def _tc_first(degp, xp, W0):
    return pl.pallas_call(
        _tc_first_body,
        grid=(_NB,),
        in_specs=[
            pl.BlockSpec((128, 16), lambda i: (i, 0)),
            pl.BlockSpec((128, 16), lambda i: (i + _NB, 0)),
            pl.BlockSpec((128, _D), lambda i: (i, 0)),
            pl.BlockSpec((_D, _D), lambda i: (0, 0)),
        ],
        out_specs=[pl.BlockSpec((128, _D), lambda i: (i, 0))] * 2,
        out_shape=[jax.ShapeDtypeStruct((_NP, _D), jnp.float32)] * 2,
    )(degp, degp, xp, W0)


def _tc_mid_body(p0_ref, p1_ref, h_ref, dinv_ref, b_ref, w_ref, o_ref):
    pre = dinv_ref[...] * (p0_ref[...] + p1_ref[...] + h_ref[...]) + b_ref[...]
    a = jnp.tanh(pre)
    o_ref[...] = dinv_ref[...] * jnp.dot(a, w_ref[...],
                                         preferred_element_type=jnp.float32)


def _tc_mid(p, h_prev, dinv, b, W):
    return pl.pallas_call(
        _tc_mid_body,
        grid=(_NB,),
        in_specs=[
            pl.BlockSpec((128, _D), lambda i: (i, 0)),
            pl.BlockSpec((128, _D), lambda i: (i + _NB, 0)),
            pl.BlockSpec((128, _D), lambda i: (i, 0)),
            pl.BlockSpec((128, _D), lambda i: (i, 0)),
            pl.BlockSpec((1, _D), lambda i: (0, 0)),
            pl.BlockSpec((_D, _D), lambda i: (0, 0)),
        ],
        out_specs=pl.BlockSpec((128, _D), lambda i: (i, 0)),
        out_shape=jax.ShapeDtypeStruct((_NP, _D), jnp.float32),
    )(p, p, h_prev, dinv, b.reshape(1, _D), W)


def _tc_final_body(p0_ref, p1_ref, h_ref, dinv_ref, b_ref, o_ref):
    pre = dinv_ref[...] * (p0_ref[...] + p1_ref[...] + h_ref[...]) + b_ref[...]
    o_ref[...] = jnp.tanh(pre)


def _tc_final(p, h_prev, dinv, b):
    return pl.pallas_call(
        _tc_final_body,
        grid=(_NB,),
        in_specs=[
            pl.BlockSpec((128, _D), lambda i: (i, 0)),
            pl.BlockSpec((128, _D), lambda i: (i + _NB, 0)),
            pl.BlockSpec((128, _D), lambda i: (i, 0)),
            pl.BlockSpec((128, _D), lambda i: (i, 0)),
            pl.BlockSpec((1, _D), lambda i: (0, 0)),
        ],
        out_specs=pl.BlockSpec((128, _D), lambda i: (i, 0)),
        out_shape=jax.ShapeDtypeStruct((_NP, _D), jnp.float32),
    )(p, p, h_prev, dinv, b.reshape(1, _D))


def kernel(x, edge_index, W0, b0, W1, b1, W2, b2, W3, b3):
    n = x.shape[0]
    e = edge_index.shape[1]
    npad = _NP - n
    epad = _EP - e
    pad_src = jnp.arange(epad, dtype=jnp.int32) % n
    pad_dst = n + (jnp.arange(epad, dtype=jnp.int32) % npad)
    src_pad = jnp.concatenate([edge_index[0], pad_src])
    dst_pad = jnp.concatenate([edge_index[1], pad_dst])
    xp = jnp.pad(x, ((0, npad), (0, 0)))

    ones_rows = jnp.ones((_CHUNK, 16), jnp.float32)
    zero16 = jnp.zeros((_CHUNK, 16), jnp.float32)
    zero_rows = jnp.zeros((_CHUNK, _D), jnp.float32)
    iota1 = jnp.arange(_NP, dtype=jnp.int32)

    degp = _sc_hist(dst_pad, ones_rows, zero16, iota1)
    dinv, h = _tc_first(degp, xp, W0)
    p = _sc_propagate(h, src_pad, dst_pad, zero_rows, iota1)
    h = _tc_mid(p, h, dinv, b0, W1)
    p = _sc_propagate(h, src_pad, dst_pad, zero_rows, iota1)
    h = _tc_mid(p, h, dinv, b1, W2)
    p = _sc_propagate(h, src_pad, dst_pad, zero_rows, iota1)
    h = _tc_mid(p, h, dinv, b2, W3)
    p = _sc_propagate(h, src_pad, dst_pad, zero_rows, iota1)
    out = _tc_final(p, h, dinv, b3)
    return out[:n]


# speedup vs baseline: 12.2169x; 1.1537x over previous
import functools

import jax
import jax.numpy as jnp
from jax import lax
from jax.experimental import pallas as pl
from jax.experimental.pallas import tpu as pltpu
from jax.experimental.pallas import tpu_sc as plsc

_N = 10000
_NP = 10240
_D = 128
_CHUNK = 128
_CPT = 80
_EP = 32 * _CPT * _CHUNK
_RPT = _NP // 16
_NB = _NP // 128


def _sc_mesh():
    return plsc.VectorSubcoreMesh(core_axis_name="c", subcore_axis_name="s",
                                  num_cores=2, num_subcores=16)


_KPT = _RPT // _CHUNK


def _hist_body(dst_hbm, ones_hbm, zero_hbm, iota_hbm, out_hbm,
               didx, ones_v, zrows_v, gidx, rowbuf, deg):
    c = lax.axis_index("c")
    s = lax.axis_index("s")
    w = c * 16 + s
    pltpu.sync_copy(ones_hbm, ones_v)
    pltpu.sync_copy(zero_hbm, zrows_v)
    for k in range(_KPT):
        pltpu.sync_copy(
            iota_hbm.at[pl.ds(s * _RPT + k * _CHUNK, _CHUNK)], gidx)
        pltpu.sync_copy(zrows_v, deg.at[gidx])
    plsc.subcore_barrier()

    def step(j, carry):
        base = (w * _CPT + j) * _CHUNK
        pltpu.sync_copy(dst_hbm.at[pl.ds(base, _CHUNK)], didx)
        pltpu.sync_copy(ones_v, deg.at[didx], add=True)
        return carry

    lax.fori_loop(0, _CPT, step, 0)
    plsc.subcore_barrier()
    for k in range(_KPT):
        pltpu.sync_copy(
            iota_hbm.at[pl.ds(s * _RPT + k * _CHUNK, _CHUNK)], gidx)
        pltpu.sync_copy(deg.at[gidx], rowbuf)
        pltpu.sync_copy(
            rowbuf, out_hbm.at[pl.ds(c * _NP + s * _RPT + k * _CHUNK, _CHUNK)])


def _sc_hist(dst_pad, ones_rows, zero_rows, iota1):
    return pl.kernel(
        _hist_body,
        out_type=jax.ShapeDtypeStruct((2 * _NP, 16), jnp.float32),
        mesh=_sc_mesh(),
        scratch_types=[
            pltpu.VMEM((_CHUNK,), jnp.int32),
            pltpu.VMEM((_CHUNK, 16), jnp.float32),
            pltpu.VMEM((_CHUNK, 16), jnp.float32),
            pltpu.VMEM((_CHUNK,), jnp.int32),
            pltpu.VMEM((_CHUNK, 16), jnp.float32),
            pltpu.VMEM_SHARED((_NP, 16), jnp.float32),
        ],
    )(dst_pad, ones_rows, zero_rows, iota1)


def _prop_body(h_hbm, src_hbm, dst_hbm, zero_hbm, iota_hbm, out_hbm,
               sidx0, sidx1, didx0, didx1, gidx, rows0, rows1, acc,
               sem0, sem1):
    c = lax.axis_index("c")
    s = lax.axis_index("s")
    w = c * 16 + s
    sidx = (sidx0, sidx1)
    didx = (didx0, didx1)
    rows = (rows0, rows1)
    sem = (sem0, sem1)
    pltpu.sync_copy(zero_hbm, rows0)
    for k in range(_KPT):
        pltpu.sync_copy(
            iota_hbm.at[pl.ds(s * _RPT + k * _CHUNK, _CHUNK)], gidx)
        pltpu.sync_copy(rows0, acc.at[gidx])
    plsc.subcore_barrier()

    def stage(j, b, first):
        base = (w * _CPT + j) * _CHUNK
        if not first:
            pltpu.make_async_copy(rows[b], acc.at[didx[b]], sem[b]).wait()
        pltpu.sync_copy(src_hbm.at[pl.ds(base, _CHUNK)], sidx[b])
        pltpu.sync_copy(dst_hbm.at[pl.ds(base, _CHUNK)], didx[b])
        pltpu.sync_copy(h_hbm.at[sidx[b]], rows[b])
        pltpu.async_copy(rows[b], acc.at[didx[b]], sem[b], add=True)

    stage(0, 0, True)
    stage(1, 1, True)

    def pair(k, carry):
        stage(2 * k + 2, 0, False)
        stage(2 * k + 3, 1, False)
        return carry

    lax.fori_loop(0, (_CPT - 2) // 2, pair, 0)
    pltpu.make_async_copy(rows0, acc.at[didx0], sem0).wait()
    pltpu.make_async_copy(rows1, acc.at[didx1], sem1).wait()
    plsc.subcore_barrier()
    for k in range(_KPT):
        r = s * _RPT + k * _CHUNK
        pltpu.sync_copy(iota_hbm.at[pl.ds(r, _CHUNK)], gidx)
        pltpu.sync_copy(acc.at[gidx], rows0)
        pltpu.sync_copy(rows0, out_hbm.at[pl.ds(c * _NP + r, _CHUNK)])


def _sc_propagate(h, src_pad, dst_pad, zero_rows, iota1):
    return pl.kernel(
        _prop_body,
        out_type=jax.ShapeDtypeStruct((2 * _NP, _D), jnp.float32),
        mesh=_sc_mesh(),
        scratch_types=[
            pltpu.VMEM((_CHUNK,), jnp.int32),
            pltpu.VMEM((_CHUNK,), jnp.int32),
            pltpu.VMEM((_CHUNK,), jnp.int32),
            pltpu.VMEM((_CHUNK,), jnp.int32),
            pltpu.VMEM((_CHUNK,), jnp.int32),
            pltpu.VMEM((_CHUNK, _D), jnp.float32),
            pltpu.VMEM((_CHUNK, _D), jnp.float32),
            pltpu.VMEM_SHARED((_NP, _D), jnp.float32),
            pltpu.SemaphoreType.DMA,
            pltpu.SemaphoreType.DMA,
        ],
    )(h, src_pad, dst_pad, zero_rows, iota1)


def _tc_first_body(dega_ref, degb_ref, x_ref, w_ref, dinv_ref, h_ref):
    i = pl.program_id(0)
    deg = dega_ref[:, 0:1] + degb_ref[:, 0:1] + 1.0
    row = lax.broadcasted_iota(jnp.int32, (128, 1), 0) + i * 128
    dinv = jnp.where(row < _N, lax.rsqrt(deg), 0.0)
    dinv_b = jnp.broadcast_to(dinv, (128, _D))
    dinv_ref[...] = dinv_b
    h_ref[...] = dinv_b * jnp.dot(x_ref[...], w_ref[...],
                                  preferred_element_type=jnp.float32)


def _tc_first(degp, xp, W0):
    return pl.pallas_call(
        _tc_first_body,
        grid=(_NB,),
        in_specs=[
            pl.BlockSpec((128, 16), lambda i: (i, 0)),
            pl.BlockSpec((128, 16), lambda i: (i + _NB, 0)),
            pl.BlockSpec((128, _D), lambda i: (i, 0)),
            pl.BlockSpec((_D, _D), lambda i: (0, 0)),
        ],
        out_specs=[pl.BlockSpec((128, _D), lambda i: (i, 0))] * 2,
        out_shape=[jax.ShapeDtypeStruct((_NP, _D), jnp.float32)] * 2,
    )(degp, degp, xp, W0)


def _tc_mid_body(p0_ref, p1_ref, h_ref, dinv_ref, b_ref, w_ref, o_ref):
    pre = dinv_ref[...] * (p0_ref[...] + p1_ref[...] + h_ref[...]) + b_ref[...]
    a = jnp.tanh(pre)
    o_ref[...] = dinv_ref[...] * jnp.dot(a, w_ref[...],
                                         preferred_element_type=jnp.float32)


def _tc_mid(p, h_prev, dinv, b, W):
    return pl.pallas_call(
        _tc_mid_body,
        grid=(_NB,),
        in_specs=[
            pl.BlockSpec((128, _D), lambda i: (i, 0)),
            pl.BlockSpec((128, _D), lambda i: (i + _NB, 0)),
            pl.BlockSpec((128, _D), lambda i: (i, 0)),
            pl.BlockSpec((128, _D), lambda i: (i, 0)),
            pl.BlockSpec((1, _D), lambda i: (0, 0)),
            pl.BlockSpec((_D, _D), lambda i: (0, 0)),
        ],
        out_specs=pl.BlockSpec((128, _D), lambda i: (i, 0)),
        out_shape=jax.ShapeDtypeStruct((_NP, _D), jnp.float32),
    )(p, p, h_prev, dinv, b.reshape(1, _D), W)


def _tc_final_body(p0_ref, p1_ref, h_ref, dinv_ref, b_ref, o_ref):
    pre = dinv_ref[...] * (p0_ref[...] + p1_ref[...] + h_ref[...]) + b_ref[...]
    o_ref[...] = jnp.tanh(pre)


def _tc_final(p, h_prev, dinv, b):
    return pl.pallas_call(
        _tc_final_body,
        grid=(_NB,),
        in_specs=[
            pl.BlockSpec((128, _D), lambda i: (i, 0)),
            pl.BlockSpec((128, _D), lambda i: (i + _NB, 0)),
            pl.BlockSpec((128, _D), lambda i: (i, 0)),
            pl.BlockSpec((128, _D), lambda i: (i, 0)),
            pl.BlockSpec((1, _D), lambda i: (0, 0)),
        ],
        out_specs=pl.BlockSpec((128, _D), lambda i: (i, 0)),
        out_shape=jax.ShapeDtypeStruct((_NP, _D), jnp.float32),
    )(p, p, h_prev, dinv, b.reshape(1, _D))


def kernel(x, edge_index, W0, b0, W1, b1, W2, b2, W3, b3):
    n = x.shape[0]
    e = edge_index.shape[1]
    npad = _NP - n
    epad = _EP - e
    pad_src = jnp.arange(epad, dtype=jnp.int32) % n
    pad_dst = n + (jnp.arange(epad, dtype=jnp.int32) % npad)
    src_pad = jnp.concatenate([edge_index[0], pad_src])
    dst_pad = jnp.concatenate([edge_index[1], pad_dst])
    xp = jnp.pad(x, ((0, npad), (0, 0)))

    ones_rows = jnp.ones((_CHUNK, 16), jnp.float32)
    zero16 = jnp.zeros((_CHUNK, 16), jnp.float32)
    zero_rows = jnp.zeros((_CHUNK, _D), jnp.float32)
    iota1 = jnp.arange(_NP, dtype=jnp.int32)

    degp = _sc_hist(dst_pad, ones_rows, zero16, iota1)
    dinv, h = _tc_first(degp, xp, W0)
    p = _sc_propagate(h, src_pad, dst_pad, zero_rows, iota1)
    h = _tc_mid(p, h, dinv, b0, W1)
    p = _sc_propagate(h, src_pad, dst_pad, zero_rows, iota1)
    h = _tc_mid(p, h, dinv, b1, W2)
    p = _sc_propagate(h, src_pad, dst_pad, zero_rows, iota1)
    h = _tc_mid(p, h, dinv, b2, W3)
    p = _sc_propagate(h, src_pad, dst_pad, zero_rows, iota1)
    out = _tc_final(p, h, dinv, b3)
    return out[:n]


# speedup vs baseline: 16.1116x; 1.3188x over previous
import functools

import jax
import jax.numpy as jnp
from jax import lax
from jax.experimental import pallas as pl
from jax.experimental.pallas import tpu as pltpu
from jax.experimental.pallas import tpu_sc as plsc

_N = 10000
_NP = 10240
_D = 128
_CHUNK = 128
_CPT = 80
_EP = 32 * _CPT * _CHUNK
_RPT = _NP // 16
_NB = _NP // 128


def _sc_mesh():
    return plsc.VectorSubcoreMesh(core_axis_name="c", subcore_axis_name="s",
                                  num_cores=2, num_subcores=16)


_KPT = _RPT // _CHUNK


def _hist_body(dst_hbm, ones_hbm, zero_hbm, iota_hbm, out_hbm,
               didx, ones_v, zrows_v, gidx, rowbuf, deg):
    c = lax.axis_index("c")
    s = lax.axis_index("s")
    w = c * 16 + s
    pltpu.sync_copy(ones_hbm, ones_v)
    pltpu.sync_copy(zero_hbm, zrows_v)
    for k in range(_KPT):
        pltpu.sync_copy(
            iota_hbm.at[pl.ds(s * _RPT + k * _CHUNK, _CHUNK)], gidx)
        pltpu.sync_copy(zrows_v, deg.at[gidx])
    plsc.subcore_barrier()

    def step(j, carry):
        base = (w * _CPT + j) * _CHUNK
        pltpu.sync_copy(dst_hbm.at[pl.ds(base, _CHUNK)], didx)
        pltpu.sync_copy(ones_v, deg.at[didx], add=True)
        return carry

    lax.fori_loop(0, _CPT, step, 0)
    plsc.subcore_barrier()
    for k in range(_KPT):
        pltpu.sync_copy(
            iota_hbm.at[pl.ds(s * _RPT + k * _CHUNK, _CHUNK)], gidx)
        pltpu.sync_copy(deg.at[gidx], rowbuf)
        pltpu.sync_copy(
            rowbuf, out_hbm.at[pl.ds(c * _NP + s * _RPT + k * _CHUNK, _CHUNK)])


def _sc_hist(dst_pad, ones_rows, zero_rows, iota1):
    return pl.kernel(
        _hist_body,
        out_type=jax.ShapeDtypeStruct((2 * _NP, 16), jnp.float32),
        mesh=_sc_mesh(),
        scratch_types=[
            pltpu.VMEM((_CHUNK,), jnp.int32),
            pltpu.VMEM((_CHUNK, 16), jnp.float32),
            pltpu.VMEM((_CHUNK, 16), jnp.float32),
            pltpu.VMEM((_CHUNK,), jnp.int32),
            pltpu.VMEM((_CHUNK, 16), jnp.float32),
            pltpu.VMEM_SHARED((_NP, 16), jnp.float32),
        ],
    )(dst_pad, ones_rows, zero_rows, iota1)


_DH = _D // 2


def _prop_body(h_hbm, src_hbm, dst_hbm, zero_hbm, iota_hbm, out_hbm,
               sidx, didx0, didx1, gidx, rows0, rows1, hsp, acc,
               sems0, sems1):
    c = lax.axis_index("c")
    s = lax.axis_index("s")
    w = c * 16 + s
    didx = (didx0, didx1)
    rows = (rows0, rows1)
    sems = (sems0, sems1)
    fo = c * _DH
    pltpu.sync_copy(zero_hbm.at[pl.ds(0, _CHUNK), pl.ds(0, _DH)], rows0)
    for k in range(_KPT):
        r = s * _RPT + k * _CHUNK
        pltpu.sync_copy(iota_hbm.at[pl.ds(r, _CHUNK)], gidx)
        pltpu.sync_copy(rows0, acc.at[gidx])
        pltpu.sync_copy(h_hbm.at[pl.ds(r, _CHUNK), pl.ds(fo, _DH)], rows1)
        pltpu.sync_copy(rows1, hsp.at[gidx])
    plsc.subcore_barrier()

    def drain_scat(b):
        pltpu.make_async_copy(rows[b], acc.at[didx[b]], sems[b]).wait()

    def stage(j, b, first):
        base = (w * _CPT + j) * _CHUNK
        if not first:
            drain_scat(b)
        pltpu.sync_copy(src_hbm.at[pl.ds(base, _CHUNK)], sidx)
        pltpu.sync_copy(dst_hbm.at[pl.ds(base, _CHUNK)], didx[b])
        pltpu.sync_copy(hsp.at[sidx], rows[b])
        pltpu.async_copy(rows[b], acc.at[didx[b]], sems[b], add=True)

    stage(0, 0, True)
    stage(1, 1, True)

    def pair(k, carry):
        stage(2 * k + 2, 0, False)
        stage(2 * k + 3, 1, False)
        return carry

    lax.fori_loop(0, (_CPT - 2) // 2, pair, 0)
    drain_scat(0)
    drain_scat(1)
    plsc.subcore_barrier()
    for k in range(_KPT):
        r = s * _RPT + k * _CHUNK
        pltpu.sync_copy(iota_hbm.at[pl.ds(r, _CHUNK)], gidx)
        pltpu.sync_copy(acc.at[gidx], rows0)
        pltpu.sync_copy(rows0,
                        out_hbm.at[pl.ds(r, _CHUNK), pl.ds(fo, _DH)])


def _sc_propagate(h, src_pad, dst_pad, zero_rows, iota1):
    return pl.kernel(
        _prop_body,
        out_type=jax.ShapeDtypeStruct((_NP, _D), jnp.float32),
        mesh=_sc_mesh(),
        compiler_params=pltpu.CompilerParams(use_tc_tiling_on_sc=False),
        scratch_types=(
            [pltpu.VMEM((_CHUNK,), jnp.int32)] * 4
            + [pltpu.VMEM((_CHUNK, _DH), jnp.float32)] * 2
            + [pltpu.VMEM_SHARED((_NP, _DH), jnp.float32)] * 2
            + [pltpu.SemaphoreType.DMA] * 2
        ),
    )(h, src_pad, dst_pad, zero_rows, iota1)


def _tc_first_body(dega_ref, degb_ref, x_ref, w_ref, dinv_ref, h_ref):
    i = pl.program_id(0)
    deg = dega_ref[:, 0:1] + degb_ref[:, 0:1] + 1.0
    row = lax.broadcasted_iota(jnp.int32, (128, 1), 0) + i * 128
    dinv = jnp.where(row < _N, lax.rsqrt(deg), 0.0)
    dinv_b = jnp.broadcast_to(dinv, (128, _D))
    dinv_ref[...] = dinv_b
    h_ref[...] = dinv_b * jnp.dot(x_ref[...], w_ref[...],
                                  preferred_element_type=jnp.float32)


def _tc_first(degp, xp, W0):
    return pl.pallas_call(
        _tc_first_body,
        grid=(_NB,),
        in_specs=[
            pl.BlockSpec((128, 16), lambda i: (i, 0)),
            pl.BlockSpec((128, 16), lambda i: (i + _NB, 0)),
            pl.BlockSpec((128, _D), lambda i: (i, 0)),
            pl.BlockSpec((_D, _D), lambda i: (0, 0)),
        ],
        out_specs=[pl.BlockSpec((128, _D), lambda i: (i, 0))] * 2,
        out_shape=[jax.ShapeDtypeStruct((_NP, _D), jnp.float32)] * 2,
    )(degp, degp, xp, W0)


def _tc_mid_body(p_ref, h_ref, dinv_ref, b_ref, w_ref, o_ref):
    pre = dinv_ref[...] * (p_ref[...] + h_ref[...]) + b_ref[...]
    a = jnp.tanh(pre)
    o_ref[...] = dinv_ref[...] * jnp.dot(a, w_ref[...],
                                         preferred_element_type=jnp.float32)


def _tc_mid(p, h_prev, dinv, b, W):
    return pl.pallas_call(
        _tc_mid_body,
        grid=(_NB,),
        in_specs=[
            pl.BlockSpec((128, _D), lambda i: (i, 0)),
            pl.BlockSpec((128, _D), lambda i: (i, 0)),
            pl.BlockSpec((128, _D), lambda i: (i, 0)),
            pl.BlockSpec((1, _D), lambda i: (0, 0)),
            pl.BlockSpec((_D, _D), lambda i: (0, 0)),
        ],
        out_specs=pl.BlockSpec((128, _D), lambda i: (i, 0)),
        out_shape=jax.ShapeDtypeStruct((_NP, _D), jnp.float32),
    )(p, h_prev, dinv, b.reshape(1, _D), W)


def _tc_final_body(p_ref, h_ref, dinv_ref, b_ref, o_ref):
    pre = dinv_ref[...] * (p_ref[...] + h_ref[...]) + b_ref[...]
    o_ref[...] = jnp.tanh(pre)


def _tc_final(p, h_prev, dinv, b):
    return pl.pallas_call(
        _tc_final_body,
        grid=(_NB,),
        in_specs=[
            pl.BlockSpec((128, _D), lambda i: (i, 0)),
            pl.BlockSpec((128, _D), lambda i: (i, 0)),
            pl.BlockSpec((128, _D), lambda i: (i, 0)),
            pl.BlockSpec((1, _D), lambda i: (0, 0)),
        ],
        out_specs=pl.BlockSpec((128, _D), lambda i: (i, 0)),
        out_shape=jax.ShapeDtypeStruct((_NP, _D), jnp.float32),
    )(p, h_prev, dinv, b.reshape(1, _D))


def kernel(x, edge_index, W0, b0, W1, b1, W2, b2, W3, b3):
    n = x.shape[0]
    e = edge_index.shape[1]
    npad = _NP - n
    epad = _EP - e
    pad_src = jnp.arange(epad, dtype=jnp.int32) % n
    pad_dst = n + (jnp.arange(epad, dtype=jnp.int32) % npad)
    src_pad = jnp.concatenate([edge_index[0], pad_src])
    dst_pad = jnp.concatenate([edge_index[1], pad_dst])
    xp = jnp.pad(x, ((0, npad), (0, 0)))

    ones_rows = jnp.ones((_CHUNK, 16), jnp.float32)
    zero16 = jnp.zeros((_CHUNK, 16), jnp.float32)
    zero_rows = jnp.zeros((_CHUNK, _D), jnp.float32)
    iota1 = jnp.arange(_NP, dtype=jnp.int32)

    def prop(h):
        return _sc_propagate(h, src_pad, dst_pad, zero_rows, iota1)

    degp = _sc_hist(dst_pad, ones_rows, zero16, iota1)
    dinv, h = _tc_first(degp, xp, W0)
    p = prop(h)
    h = _tc_mid(p, h, dinv, b0, W1)
    p = prop(h)
    h = _tc_mid(p, h, dinv, b1, W2)
    p = prop(h)
    h = _tc_mid(p, h, dinv, b2, W3)
    p = prop(h)
    out = _tc_final(p, h, dinv, b3)
    return out[:n]
